# Initial kernel scaffold; baseline (speedup 1.0000x reference)
import jax
import jax.numpy as jnp
from jax.experimental import pallas as pl


def kernel(x, Wr, ln_g, ln_b, W1, b1, W2, b2):
    raise NotImplementedError("write your pallas kernel here")


# speedup vs baseline: 1.0296x; 1.0296x over previous
import functools

import jax
import jax.numpy as jnp
from jax import lax
from jax.experimental import pallas as pl
from jax.experimental.pallas import tpu as pltpu

try:
    from jax.experimental.pallas import tpu_sc as plsc
    _HAS_SC = True
except ImportError:
    plsc = None
    _HAS_SC = False

DIM = 1024
MLP = 4096
E = 8
TOPK = 2
BLK = 256
G = 16 + E
P = G * BLK

NW = 32
_NC = 2


def _ffn_block(be_ref, bv_ref, xg_ref, gate_ref, lng_ref, lnb_ref,
               w1_ref, b1_ref, w2_ref, b2_ref, y_ref):
    g = pl.program_id(0)

    @pl.when(bv_ref[g] != 0)
    def _():
        xg = xg_ref[...]
        mu = jnp.mean(xg, axis=-1, keepdims=True)
        var = jnp.mean((xg - mu) ** 2, axis=-1, keepdims=True)
        xn = (xg - mu) / jnp.sqrt(var + 1e-5) * lng_ref[0] + lnb_ref[0]
        h = jnp.dot(xn.astype(jnp.bfloat16), w1_ref[0],
                    preferred_element_type=jnp.float32) + b1_ref[0]
        h = jax.nn.gelu(h)
        o = jnp.dot(h.astype(jnp.bfloat16), w2_ref[0],
                    preferred_element_type=jnp.float32) + b2_ref[0]
        y_ref[...] = (xg + o) * gate_ref[...]


def _grouped_ffn(xg, row_gate, ln_g, ln_b, w1b, b1, w2b, b2,
                 block_expert, block_valid):
    grid_spec = pltpu.PrefetchScalarGridSpec(
        num_scalar_prefetch=2,
        grid=(G,),
        in_specs=[
            pl.BlockSpec((BLK, DIM), lambda g, be, bv: (g, 0)),
            pl.BlockSpec((BLK, 1), lambda g, be, bv: (g, 0)),
            pl.BlockSpec((1, 1, DIM), lambda g, be, bv: (be[g], 0, 0)),
            pl.BlockSpec((1, 1, DIM), lambda g, be, bv: (be[g], 0, 0)),
            pl.BlockSpec((1, DIM, MLP), lambda g, be, bv: (be[g], 0, 0)),
            pl.BlockSpec((1, 1, MLP), lambda g, be, bv: (be[g], 0, 0)),
            pl.BlockSpec((1, MLP, DIM), lambda g, be, bv: (be[g], 0, 0)),
            pl.BlockSpec((1, 1, DIM), lambda g, be, bv: (be[g], 0, 0)),
        ],
        out_specs=pl.BlockSpec((BLK, DIM), lambda g, be, bv: (g, 0)),
    )
    return pl.pallas_call(
        _ffn_block,
        grid_spec=grid_spec,
        out_shape=jax.ShapeDtypeStruct((P, DIM), jnp.float32),
    )(block_expert, block_valid, xg, row_gate.reshape(P, 1),
      ln_g.reshape(E, 1, DIM), ln_b.reshape(E, 1, DIM), w1b,
      b1.reshape(E, 1, MLP), w2b, b2.reshape(E, 1, DIM))


def _sc_gather(xf, row_token):
    rpw = P // NW
    ch = 64
    nch = rpw // ch
    mesh = plsc.VectorSubcoreMesh(core_axis_name="c", subcore_axis_name="s")

    @functools.partial(
        pl.kernel, mesh=mesh,
        out_type=jax.ShapeDtypeStruct((P, DIM), jnp.float32),
        scratch_types=[
            pltpu.VMEM((ch,), jnp.int32),
            pltpu.VMEM((ch, DIM), jnp.float32),
            pltpu.SemaphoreType.DMA,
        ],
    )
    def k(x_hbm, tok_hbm, out_hbm, idx_v, rows_v, sem):
        wid = lax.axis_index("s") * _NC + lax.axis_index("c")
        for c in range(nch):
            base = wid * rpw + c * ch
            pltpu.sync_copy(tok_hbm.at[pl.ds(base, ch)], idx_v)
            pltpu.async_copy(x_hbm.at[idx_v], rows_v, sem).wait()
            pltpu.sync_copy(rows_v, out_hbm.at[pl.ds(base, ch)])

    return k(xf, row_token)


def _sc_combine(y, pos0, pos1, n_tokens):
    rpw = n_tokens // NW
    ch = 32
    nch = rpw // ch
    mesh = plsc.VectorSubcoreMesh(core_axis_name="c", subcore_axis_name="s")

    @functools.partial(
        pl.kernel, mesh=mesh,
        out_type=jax.ShapeDtypeStruct((n_tokens, DIM), jnp.float32),
        scratch_types=[
            pltpu.VMEM((ch,), jnp.int32),
            pltpu.VMEM((ch,), jnp.int32),
            pltpu.VMEM((ch, DIM), jnp.float32),
            pltpu.VMEM((ch, DIM), jnp.float32),
            pltpu.SemaphoreType.DMA,
        ],
    )
    def k(y_hbm, p0_hbm, p1_hbm, out_hbm, i0_v, i1_v, a_v, b_v, sem):
        wid = lax.axis_index("s") * _NC + lax.axis_index("c")
        for c in range(nch):
            base = wid * rpw + c * ch
            pltpu.sync_copy(p0_hbm.at[pl.ds(base, ch)], i0_v)
            pltpu.sync_copy(p1_hbm.at[pl.ds(base, ch)], i1_v)
            pltpu.async_copy(y_hbm.at[i0_v], a_v, sem).wait()
            pltpu.async_copy(y_hbm.at[i1_v], b_v, sem).wait()

            def row_add(r, _):
                def lane_add(j, _):
                    s = pl.ds(j * 16, 16)
                    a_v[r, s] = a_v[r, s] + b_v[r, s]
                    return 0
                return lax.fori_loop(0, DIM // 16, lane_add, 0)

            lax.fori_loop(0, ch, row_add, 0)
            pltpu.sync_copy(a_v, out_hbm.at[pl.ds(base, ch)])

    return k(y, pos0, pos1)


def _route(xf, Wr):
    n = xf.shape[0]
    nk = n * TOPK
    logits = xf @ Wr
    probs = jax.nn.softmax(logits, axis=-1)
    topv, topi = jax.lax.top_k(probs, TOPK)
    gates = topv / (jnp.sum(topv, axis=-1, keepdims=True) + 1e-9)

    e_flat = topi.reshape(-1).astype(jnp.int32)
    order = jnp.argsort(e_flat, stable=True).astype(jnp.int32)
    e_sorted = e_flat[order]
    counts = jnp.bincount(e_flat, length=E).astype(jnp.int32)
    blocks_e = (counts + BLK - 1) // BLK
    bcum = jnp.cumsum(blocks_e)
    padded_start = (bcum - blocks_e) * BLK
    csum = jnp.cumsum(counts) - counts
    ranks = jnp.arange(nk, dtype=jnp.int32) - csum[e_sorted]
    p_sorted = (padded_start[e_sorted] + ranks).astype(jnp.int32)
    tok_sorted = (order // TOPK).astype(jnp.int32)

    row_token = jnp.zeros((P,), jnp.int32).at[p_sorted].set(tok_sorted)
    row_gate = jnp.zeros((P,), jnp.float32).at[p_sorted].set(
        gates.reshape(-1)[order])
    pos = jnp.zeros((nk,), jnp.int32).at[order].set(p_sorted).reshape(n, TOPK)

    garr = jnp.arange(G, dtype=jnp.int32)
    block_expert_raw = jnp.searchsorted(bcum, garr, side="right").astype(jnp.int32)
    last_e = jnp.argmax(jnp.where(counts > 0,
                                  jnp.arange(E, dtype=jnp.int32), -1)).astype(jnp.int32)
    block_expert = jnp.minimum(block_expert_raw, last_e)
    block_valid = (garr < bcum[-1]).astype(jnp.int32)
    return row_token, row_gate, pos, block_expert, block_valid


def kernel(x, Wr, ln_g, ln_b, W1, b1, W2, b2):
    b, t, d = x.shape
    n = b * t
    xf = x.reshape(n, d)

    row_token, row_gate, pos, block_expert, block_valid = _route(xf, Wr)

    xg = _sc_gather(xf, row_token)

    w1b = W1.astype(jnp.bfloat16)
    w2b = W2.astype(jnp.bfloat16)
    y = _grouped_ffn(xg, row_gate, ln_g, ln_b, w1b, b1, w2b, b2,
                     block_expert, block_valid)

    out = _sc_combine(y, pos[:, 0], pos[:, 1], n)
    return out.reshape(b, t, d)
